# baseline (speedup 1.0000x reference)
import functools

import jax
import jax.numpy as jnp
from jax.experimental import pallas as pl


def _mlp_block_kernel(x_ref, w1_ref, b1_ref, w2_ref, b2_ref, o_ref):
    h = jnp.dot(x_ref[...], w1_ref[...], preferred_element_type=jnp.float32)
    h = jnp.maximum(h + b1_ref[...], 0.0)
    o = jnp.dot(h, w2_ref[...], preferred_element_type=jnp.float32)
    o_ref[...] = o + b2_ref[...]


@functools.partial(jax.jit, static_argnums=(5,))
def _mlp_pallas(x, w1, b1, w2, b2, bm):
    m, i = x.shape
    h = w1.shape[1]
    o = w2.shape[1]
    mp = ((m + bm - 1) // bm) * bm
    if mp != m:
        x = jnp.pad(x, ((0, mp - m), (0, 0)))
    out = pl.pallas_call(
        _mlp_block_kernel,
        grid=(mp // bm,),
        in_specs=[
            pl.BlockSpec((bm, i), lambda g: (g, 0)),
            pl.BlockSpec((i, h), lambda g: (0, 0)),
            pl.BlockSpec((1, h), lambda g: (0, 0)),
            pl.BlockSpec((h, o), lambda g: (0, 0)),
            pl.BlockSpec((1, o), lambda g: (0, 0)),
        ],
        out_specs=pl.BlockSpec((bm, o), lambda g: (g, 0)),
        out_shape=jax.ShapeDtypeStruct((mp, o), jnp.float32),
    )(x, w1, b1.reshape(1, h), w2, b2.reshape(1, o))
    return out[:m]


def _mlp(p, name, x, bm):
    return _mlp_pallas(x, p[name + "_W1"], p[name + "_b1"],
                       p[name + "_W2"], p[name + "_b2"], bm)


def _scatter_mean(data, ids, num_segments):
    s = jax.ops.segment_sum(data, ids, num_segments=num_segments)
    c = jax.ops.segment_sum(jnp.ones((data.shape[0], 1), data.dtype), ids,
                            num_segments=num_segments)
    return s / jnp.clip(c, 1.0, None)


def _meta(x, edge_index, edge_attr, u, batch, params, i):
    row = edge_index[0]
    col = edge_index[1]
    eb = batch[row]
    e_in = jnp.concatenate([x[row], x[col], edge_attr, u[eb]], axis=1)
    edge_attr = _mlp(params, "e%d" % i, e_in, 4096)
    agg = _scatter_mean(edge_attr, row, x.shape[0])
    n_in = jnp.concatenate([x, agg, u[batch]], axis=1)
    x = _mlp(params, "n%d" % i, n_in, 4096)
    g_in = jnp.concatenate([u,
                            _scatter_mean(x, batch, u.shape[0]),
                            _scatter_mean(edge_attr, eb, u.shape[0])], axis=1)
    u = _mlp(params, "g%d" % i, g_in, 64)
    return x, edge_attr, u


def kernel(x, edge_index, edge_weight, u, batch, params):
    x, e, u = _meta(x, edge_index, edge_weight, u, batch, params, 1)
    x, e, u = _meta(x, edge_index, e, u, batch, params, 2)
    x, e, u = _meta(x, edge_index, e, u, batch, params, 3)
    return jax.nn.sigmoid(u)
